# baseline (speedup 1.0000x reference)
import functools
import math

import numpy as np

import jax
import jax.numpy as jnp
from jax import lax
from jax.experimental import pallas as pl
from jax.experimental.pallas import tpu as pltpu
from jax.experimental.pallas import tpu_sc as plsc

_B, _L, _H, _E, _D = 2, 4096, 16, 64, 64
_SCALES = (1, 2, 4)
_EPS = 1e-8
_KMAX = 16


def _build_dft():
    groups = []
    col = 0
    for s in _SCALES:
        Lc = _L // s
        F = Lc // 2 + 1
        Fpad = ((F + 127) // 128) * 128
        groups.append((col, F, Fpad, s))
        col += Fpad
    Ftot = ((col + 127) // 128) * 128
    FBs = []
    DI = np.zeros((Ftot, _L), np.float64)
    EI = np.zeros((Ftot, _L), np.float64)
    for (c0, F, Fpad, s) in groups:
        Lc = _L // s
        n = np.arange(Lc, dtype=np.float64)[:, None]
        k = np.arange(F, dtype=np.float64)[None, :]
        ang = 2.0 * np.pi * n * k / Lc
        CF = np.zeros((Lc, Fpad), np.float64)
        SF = np.zeros((Lc, Fpad), np.float64)
        CF[:, :F] = np.cos(ang)
        SF[:, :F] = -np.sin(ang)
        FBs.append((CF, SF))
        a = np.full((F,), 2.0)
        a[0] = 1.0
        a[-1] = 1.0
        kk = np.arange(F, dtype=np.float64)[:, None]
        nn = np.arange(Lc, dtype=np.float64)[None, :]
        ang2 = 2.0 * np.pi * kk * nn / Lc
        di0 = a[:, None] * np.cos(ang2) / Lc
        ei0 = -a[:, None] * np.sin(ang2) / Lc
        if s == 1:
            DI[c0:c0 + F, :] = di0
            EI[c0:c0 + F, :] = ei0
        else:
            i = np.arange(_L, dtype=np.float64)
            src = np.maximum((i + 0.5) * (Lc / _L) - 0.5, 0.0)
            i0 = np.clip(np.floor(src).astype(np.int64), 0, Lc - 1)
            i1 = np.clip(i0 + 1, 0, Lc - 1)
            w = src - i0
            DI[c0:c0 + F, :] = di0[:, i0] * (1.0 - w) + di0[:, i1] * w
            EI[c0:c0 + F, :] = ei0[:, i0] * (1.0 - w) + ei0[:, i1] * w
    FT = 128
    fbs = []
    for (CF, SF) in FBs:
        Lc, Fpad = CF.shape
        nt = Fpad // FT
        fbs.append(np.concatenate(
            [CF.reshape(Lc, nt, FT), SF.reshape(Lc, nt, FT)], axis=2
        ).reshape(Lc, 2 * Fpad).astype(np.float32))
    nt = Ftot // FT
    IB = np.concatenate(
        [DI.reshape(nt, FT, _L), EI.reshape(nt, FT, _L)], axis=1
    ).reshape(2 * Ftot, _L).astype(np.float32)
    return fbs, IB, groups, Ftot, FT


_FBS, _IB, _GROUPS, _FTOT, _FT = _build_dft()
_NT = _FTOT // _FT
_NT1 = _GROUPS[0][2] // _FT
_NT2 = _GROUPS[1][2] // _FT
_NT4 = _GROUPS[2][2] // _FT


def _scale_vec(scale_weights):
    sw = jax.nn.softmax(scale_weights[: len(_SCALES)])
    parts = []
    for gi, (c0, F, Fpad, s) in enumerate(_GROUPS):
        parts.append(jnp.full((Fpad,), sw[gi], jnp.float32))
    v = jnp.concatenate(parts)
    v = jnp.pad(v, (0, _FTOT - v.shape[0]))
    return v[None, :]


def _ln_kernel(q_ref, k_ref, qo_ref, ko_ref, qo2_ref, ko2_ref,
               qo4_ref, ko4_ref):
    for src, dst, dst2, dst4 in (
            (q_ref, qo_ref, qo2_ref, qo4_ref),
            (k_ref, ko_ref, ko2_ref, ko4_ref)):
        x = src[0]
        m = jnp.mean(x, axis=-1, keepdims=True)
        v = jnp.mean((x - m) ** 2, axis=-1, keepdims=True)
        y = (x - m) / jnp.sqrt(v + 1e-5)
        dst[0] = y
        yr = y.reshape(y.shape[0] // 2, 2, _H, _E)
        y2 = (yr[:, 0] + yr[:, 1]) * 0.5
        dst2[0] = y2
        y2r = y2.reshape(y2.shape[0] // 2, 2, _H, _E)
        dst4[0] = (y2r[:, 0] + y2r[:, 1]) * 0.5
    return


def _layer_norm(q, k):
    LB = 512
    grid = (_B, _L // LB)
    spec = pl.BlockSpec((1, LB, _H, _E), lambda b, lb: (b, lb, 0, 0))
    spec2 = pl.BlockSpec((1, LB // 2, _H, _E), lambda b, lb: (b, lb, 0, 0))
    spec4 = pl.BlockSpec((1, LB // 4, _H, _E), lambda b, lb: (b, lb, 0, 0))
    sds = jax.ShapeDtypeStruct
    out = pl.pallas_call(
        _ln_kernel,
        grid=grid,
        in_specs=[spec, spec],
        out_specs=[spec, spec, spec2, spec2, spec4, spec4],
        out_shape=[
            sds((_B, _L, _H, _E), jnp.float32),
            sds((_B, _L, _H, _E), jnp.float32),
            sds((_B, _L // 2, _H, _E), jnp.float32),
            sds((_B, _L // 2, _H, _E), jnp.float32),
            sds((_B, _L // 4, _H, _E), jnp.float32),
            sds((_B, _L // 4, _H, _E), jnp.float32),
        ],
    )(q, k)
    return out


_CBLK = 256
_G0 = (_B * _H * _E) // _CBLK
_HPB = _CBLK // _E


def _corr_kernel(ff_ref, qt_ref, kt_ref, qt2_ref, kt2_ref, qt4_ref, kt4_ref,
                 fb1_ref, fb2_ref, fb4_ref, ib_ref, sv_ref,
                 mean_ref, energy_ref, acc_ref, qf_ref, kf_ref):
    g1 = pl.program_id(1)
    ff = ff_ref[0, 0]

    @pl.when(g1 < _NT1)
    def _():
        qf_ref[...] = jnp.dot(qt_ref[...], fb1_ref[...],
                              preferred_element_type=jnp.float32)
        kf_ref[...] = jnp.dot(kt_ref[...], fb1_ref[...],
                              preferred_element_type=jnp.float32)

    @pl.when((g1 >= _NT1) & (g1 < _NT1 + _NT2))
    def _():
        qf_ref[...] = jnp.dot(qt2_ref[...], fb2_ref[...],
                              preferred_element_type=jnp.float32)
        kf_ref[...] = jnp.dot(kt2_ref[...], fb2_ref[...],
                              preferred_element_type=jnp.float32)

    @pl.when(g1 >= _NT1 + _NT2)
    def _():
        qf_ref[...] = jnp.dot(qt4_ref[...], fb4_ref[...],
                              preferred_element_type=jnp.float32)
        kf_ref[...] = jnp.dot(kt4_ref[...], fb4_ref[...],
                              preferred_element_type=jnp.float32)

    qf = qf_ref[...]
    kf = kf_ref[...]
    qre, qim = qf[:, :_FT], qf[:, _FT:]
    ure, uim = kf[:, :_FT] * ff, kf[:, _FT:] * ff
    mag = jnp.sqrt(ure * ure + uim * uim)
    inv = (ff * sv_ref[...]) / (mag + _EPS)
    cr = (qre * ure + qim * uim) * inv
    ci = (qim * ure - qre * uim) * inv
    contrib = jnp.dot(
        jnp.concatenate([cr, ci], axis=1), ib_ref[...],
        preferred_element_type=jnp.float32)

    @pl.when(g1 == 0)
    def _():
        acc_ref[...] = contrib

    @pl.when(g1 > 0)
    def _():
        acc_ref[...] = acc_ref[...] + contrib

    @pl.when(g1 == _NT - 1)
    def _():
        corr = jnp.clip(acc_ref[...], -10.0, 10.0)
        msum = jnp.sum(corr, axis=0, keepdims=True)
        esum = jnp.sum(corr * corr, axis=0, keepdims=True)
        g0 = pl.program_id(0)

        @pl.when(g0 % (_G0 // _B) == 0)
        def _():
            mean_ref[0] = msum
            energy_ref[0] = esum

        @pl.when(g0 % (_G0 // _B) != 0)
        def _():
            mean_ref[0] = mean_ref[0] + msum
            energy_ref[0] = energy_ref[0] + esum


def _correlate(qt, kt, qt2, kt2, qt4, kt4, svec, ff):
    grid = (_G0, _NT)
    chan = lambda g0, g1: (g0, 0)
    out = pl.pallas_call(
        _corr_kernel,
        grid=grid,
        in_specs=[
            pl.BlockSpec(memory_space=pltpu.SMEM),
            pl.BlockSpec((_CBLK, _L), chan),
            pl.BlockSpec((_CBLK, _L), chan),
            pl.BlockSpec((_CBLK, _L // 2), chan),
            pl.BlockSpec((_CBLK, _L // 2), chan),
            pl.BlockSpec((_CBLK, _L // 4), chan),
            pl.BlockSpec((_CBLK, _L // 4), chan),
            pl.BlockSpec((_L, 2 * _FT),
                         lambda g0, g1: (0, jnp.minimum(g1, _NT1 - 1))),
            pl.BlockSpec((_L // 2, 2 * _FT),
                         lambda g0, g1: (0, jnp.clip(g1 - _NT1, 0, _NT2 - 1))),
            pl.BlockSpec((_L // 4, 2 * _FT),
                         lambda g0, g1: (0, jnp.clip(g1 - _NT1 - _NT2, 0,
                                                     _NT4 - 1))),
            pl.BlockSpec((2 * _FT, _L), lambda g0, g1: (g1, 0)),
            pl.BlockSpec((1, _FT), lambda g0, g1: (0, g1)),
        ],
        out_specs=[
            pl.BlockSpec((1, 1, _L), lambda g0, g1: (g0 // (_G0 // _B), 0, 0)),
            pl.BlockSpec((1, 1, _L), lambda g0, g1: (g0 // (_G0 // _B), 0, 0)),
        ],
        out_shape=[
            jax.ShapeDtypeStruct((_B, 1, _L), jnp.float32),
            jax.ShapeDtypeStruct((_B, 1, _L), jnp.float32),
        ],
        scratch_shapes=[
            pltpu.VMEM((_CBLK, _L), jnp.float32),
            pltpu.VMEM((_CBLK, 2 * _FT), jnp.float32),
            pltpu.VMEM((_CBLK, 2 * _FT), jnp.float32),
        ],
        compiler_params=pltpu.CompilerParams(
            vmem_limit_bytes=100 * 1024 * 1024),
    )(ff, qt, kt, qt2, kt2, qt4, kt4,
      jnp.asarray(_FBS[0]), jnp.asarray(_FBS[1]), jnp.asarray(_FBS[2]),
      jnp.asarray(_IB), svec)
    return out[0][:, 0, :], out[1][:, 0, :]


_HD = _H * _D
_NW = 32
_ROWS_PER_W = (_B * _L) // _NW
_CH = 32
_NCHUNK = _ROWS_PER_W // _CH


def _agg_sc_kernel(vf_hbm, dl_hbm, nw_hbm, out_hbm,
                   dlv, nwv, idx, buf, acc, sem):
    cid = lax.axis_index("c")
    sid = lax.axis_index("s")
    wid = sid * 2 + cid
    b = wid // (_NW // _B)
    off = (wid % (_NW // _B)) * _ROWS_PER_W
    pltpu.sync_copy(dl_hbm, dlv)
    pltpu.sync_copy(nw_hbm, nwv)
    iot = lax.iota(jnp.int32, 16)
    dv = dlv[pl.ds(b * _KMAX, 16)]
    wv = nwv[pl.ds(b * _KMAX, 16)]
    out_base = pl.multiple_of(b * _L + off, _CH)

    def step(wk, dk, t0, init):
        for h in range(_CH // 16):
            rows = lax.rem(t0 + h * 16 + dk + iot, _L) + b * _L
            idx[pl.ds(h * 16, 16)] = rows
        pltpu.async_copy(vf_hbm.at[idx], buf, sem).wait()

        def body(i, _):
            def inner(j, _):
                sl = pl.ds(j * 16, 16)
                if init:
                    acc[i, sl] = wk * buf[i, sl]
                else:
                    acc[i, sl] = acc[i, sl] + wk * buf[i, sl]
                return 0
            return lax.fori_loop(0, _HD // 16, inner, 0)
        lax.fori_loop(0, _CH, body, 0)

    def chunk(c, _):
        t0 = off + c * _CH
        for k in range(_KMAX):
            wk = wv[k]
            dk = dv[k]
            if k == 0:
                step(wk, dk, t0, True)
            else:
                pl.when(wk != 0.0)(
                    functools.partial(step, wk, dk, t0, False))
        orow = pl.multiple_of(out_base + c * _CH, _CH)
        pltpu.sync_copy(acc, out_hbm.at[pl.ds(orow, _CH), :])
        return 0

    lax.fori_loop(0, _NCHUNK, chunk, 0)


def _aggregate(vf, delays, nw):
    mesh = plsc.VectorSubcoreMesh(core_axis_name="c", subcore_axis_name="s")
    kern = functools.partial(
        pl.kernel,
        mesh=mesh,
        out_type=jax.ShapeDtypeStruct((_B * _L, _HD), jnp.float32),
        scratch_types=[
            pltpu.VMEM((_B * _KMAX,), jnp.int32),
            pltpu.VMEM((_B * _KMAX,), jnp.float32),
            pltpu.VMEM((_CH,), jnp.int32),
            pltpu.VMEM((_CH, _HD), jnp.float32),
            pltpu.VMEM((_CH, _HD), jnp.float32),
            pltpu.SemaphoreType.DMA,
        ],
    )(_agg_sc_kernel)
    return kern(vf, delays, nw)


def kernel(queries, keys, values, attn_mask, scale_weights, frequency_filter):
    B, Lq, H, E = queries.shape
    D = values.shape[-1]
    qn, kn, qn2, kn2, qn4, kn4 = _layer_norm(queries, keys)

    def tr(x):
        return x.transpose(0, 2, 3, 1).reshape(B * H * E, x.shape[1])

    ff = jax.nn.sigmoid(frequency_filter[0]).reshape(1, 1)
    svec = _scale_vec(scale_weights)
    mean_sum, energy = _correlate(tr(qn), tr(kn), tr(qn2), tr(kn2),
                                  tr(qn4), tr(kn4), svec, ff)
    mean_corr = mean_sum / (H * E)

    se = jnp.sort(energy, axis=-1)[:, ::-1]
    fd = se[:, :-1] - se[:, 1:]
    sd = fd[:, :-1] - fd[:, 1:]
    elbow = jnp.argmax(sd, axis=-1) + 2
    min_k = max(2, int(0.1 * math.log(Lq)))
    max_k = min(int(0.3 * Lq), int(math.log(Lq) * 2))
    if min_k > max_k:
        max_k = min_k
    ak = jnp.clip(elbow, min_k, max_k).astype(jnp.float32)
    srt = jnp.sort(ak)
    top_k = srt[(srt.shape[0] - 1) // 2].astype(jnp.int32)

    weights, delays = lax.top_k(mean_corr, _KMAX)
    active = jnp.arange(_KMAX) < top_k
    masked = jnp.where(active[None, :], weights, -jnp.inf)
    nw = jax.nn.softmax(masked, axis=-1)

    vf = values.reshape(B * Lq, H * D)
    out = _aggregate(vf, delays.astype(jnp.int32).reshape(-1), nw.reshape(-1))
    return out.reshape(B, Lq, H, D)


# speedup vs baseline: 1.0317x; 1.0317x over previous
import functools
import math

import numpy as np

import jax
import jax.numpy as jnp
from jax import lax
from jax.experimental import pallas as pl
from jax.experimental.pallas import tpu as pltpu
from jax.experimental.pallas import tpu_sc as plsc

_B, _L, _H, _E, _D = 2, 4096, 16, 64, 64
_SCALES = (1, 2, 4)
_EPS = 1e-8
_KMAX = 16


def _build_dft():
    groups = []
    col = 0
    for s in _SCALES:
        Lc = _L // s
        F = Lc // 2 + 1
        Fpad = ((F + 127) // 128) * 128
        groups.append((col, F, Fpad, s))
        col += Fpad
    Ftot = ((col + 127) // 128) * 128
    CF = np.zeros((_L, Ftot), np.float64)
    SF = np.zeros((_L, Ftot), np.float64)
    DI = np.zeros((Ftot, _L), np.float64)
    EI = np.zeros((Ftot, _L), np.float64)
    for (c0, F, Fpad, s) in groups:
        Lc = _L // s
        n = np.arange(Lc, dtype=np.float64)[:, None]
        k = np.arange(F, dtype=np.float64)[None, :]
        ang = 2.0 * np.pi * n * k / Lc
        CF[:, c0:c0 + F] = np.repeat(np.cos(ang) / s, s, axis=0)
        SF[:, c0:c0 + F] = np.repeat(-np.sin(ang) / s, s, axis=0)
        a = np.full((F,), 2.0)
        a[0] = 1.0
        a[-1] = 1.0
        kk = np.arange(F, dtype=np.float64)[:, None]
        nn = np.arange(Lc, dtype=np.float64)[None, :]
        ang2 = 2.0 * np.pi * kk * nn / Lc
        di0 = a[:, None] * np.cos(ang2) / Lc
        ei0 = -a[:, None] * np.sin(ang2) / Lc
        if s == 1:
            DI[c0:c0 + F, :] = di0
            EI[c0:c0 + F, :] = ei0
        else:
            i = np.arange(_L, dtype=np.float64)
            src = np.maximum((i + 0.5) * (Lc / _L) - 0.5, 0.0)
            i0 = np.clip(np.floor(src).astype(np.int64), 0, Lc - 1)
            i1 = np.clip(i0 + 1, 0, Lc - 1)
            w = src - i0
            DI[c0:c0 + F, :] = di0[:, i0] * (1.0 - w) + di0[:, i1] * w
            EI[c0:c0 + F, :] = ei0[:, i0] * (1.0 - w) + ei0[:, i1] * w
    FT = 128
    nt = Ftot // FT
    FB = np.concatenate(
        [CF.reshape(_L, nt, FT), SF.reshape(_L, nt, FT)], axis=2
    ).reshape(_L, 2 * Ftot).astype(np.float32)
    IB = np.concatenate(
        [DI.reshape(nt, FT, _L), EI.reshape(nt, FT, _L)], axis=1
    ).reshape(2 * Ftot, _L).astype(np.float32)
    return FB, IB, groups, Ftot, FT


_FB, _IB, _GROUPS, _FTOT, _FT = _build_dft()
_NT = _FTOT // _FT


def _scale_vec(scale_weights):
    sw = jax.nn.softmax(scale_weights[: len(_SCALES)])
    parts = []
    for gi, (c0, F, Fpad, s) in enumerate(_GROUPS):
        parts.append(jnp.full((Fpad,), sw[gi], jnp.float32))
    v = jnp.concatenate(parts)
    v = jnp.pad(v, (0, _FTOT - v.shape[0]))
    return v[None, :]


def _ln_kernel(q_ref, k_ref, qo_ref, ko_ref):
    for src, dst in ((q_ref, qo_ref), (k_ref, ko_ref)):
        x = src[0]
        m = jnp.mean(x, axis=-1, keepdims=True)
        v = jnp.mean((x - m) ** 2, axis=-1, keepdims=True)
        dst[0] = (x - m) / jnp.sqrt(v + 1e-5)


def _layer_norm(q, k):
    LB = 512
    grid = (_B, _L // LB)
    spec = pl.BlockSpec((1, LB, _H, _E), lambda b, lb: (b, lb, 0, 0))
    out = pl.pallas_call(
        _ln_kernel,
        grid=grid,
        in_specs=[spec, spec],
        out_specs=[spec, spec],
        out_shape=[
            jax.ShapeDtypeStruct(q.shape, jnp.float32),
            jax.ShapeDtypeStruct(k.shape, jnp.float32),
        ],
    )(q, k)
    return out


_CBLK = 256
_G0 = (_B * _H * _E) // _CBLK
_HPB = _CBLK // _E


def _corr_kernel(ff_ref, qt_ref, kt_ref, fb_ref, ib_ref, sv_ref,
                 mean_ref, energy_ref, acc_ref):
    g1 = pl.program_id(1)
    ff = ff_ref[0, 0]
    qf = jnp.dot(qt_ref[...], fb_ref[...], preferred_element_type=jnp.float32)
    kf = jnp.dot(kt_ref[...], fb_ref[...], preferred_element_type=jnp.float32)
    qre, qim = qf[:, :_FT], qf[:, _FT:]
    ure, uim = kf[:, :_FT] * ff, kf[:, _FT:] * ff
    mag = jnp.sqrt(ure * ure + uim * uim)
    inv = (ff * sv_ref[...]) / (mag + _EPS)
    cr = (qre * ure + qim * uim) * inv
    ci = (qim * ure - qre * uim) * inv
    contrib = jnp.dot(
        jnp.concatenate([cr, ci], axis=1), ib_ref[...],
        preferred_element_type=jnp.float32)

    @pl.when(g1 == 0)
    def _():
        acc_ref[...] = contrib

    @pl.when(g1 > 0)
    def _():
        acc_ref[...] = acc_ref[...] + contrib

    @pl.when(g1 == _NT - 1)
    def _():
        corr = jnp.clip(acc_ref[...], -10.0, 10.0)
        msum = jnp.sum(corr, axis=0, keepdims=True)
        esum = jnp.sum(corr * corr, axis=0, keepdims=True)
        g0 = pl.program_id(0)

        @pl.when(g0 % (_G0 // _B) == 0)
        def _():
            mean_ref[0] = msum
            energy_ref[0] = esum

        @pl.when(g0 % (_G0 // _B) != 0)
        def _():
            mean_ref[0] = mean_ref[0] + msum
            energy_ref[0] = energy_ref[0] + esum


def _correlate(qt, kt, svec, ff):
    grid = (_G0, _NT)
    out = pl.pallas_call(
        _corr_kernel,
        grid=grid,
        in_specs=[
            pl.BlockSpec(memory_space=pltpu.SMEM),
            pl.BlockSpec((_CBLK, _L), lambda g0, g1: (g0, 0)),
            pl.BlockSpec((_CBLK, _L), lambda g0, g1: (g0, 0)),
            pl.BlockSpec((_L, 2 * _FT), lambda g0, g1: (0, g1)),
            pl.BlockSpec((2 * _FT, _L), lambda g0, g1: (g1, 0)),
            pl.BlockSpec((1, _FT), lambda g0, g1: (0, g1)),
        ],
        out_specs=[
            pl.BlockSpec((1, 1, _L), lambda g0, g1: (g0 // (_G0 // _B), 0, 0)),
            pl.BlockSpec((1, 1, _L), lambda g0, g1: (g0 // (_G0 // _B), 0, 0)),
        ],
        out_shape=[
            jax.ShapeDtypeStruct((_B, 1, _L), jnp.float32),
            jax.ShapeDtypeStruct((_B, 1, _L), jnp.float32),
        ],
        scratch_shapes=[pltpu.VMEM((_CBLK, _L), jnp.float32)],
    )(ff, qt, kt, jnp.asarray(_FB), jnp.asarray(_IB), svec)
    return out[0][:, 0, :], out[1][:, 0, :]


_HD = _H * _D
_NW = 32
_ROWS_PER_W = (_B * _L) // _NW
_CH = 32
_NCHUNK = _ROWS_PER_W // _CH


def _agg_sc_kernel(vf_hbm, dl_hbm, nw_hbm, out_hbm,
                   dlv, nwv, idx, buf, acc, sem):
    cid = lax.axis_index("c")
    sid = lax.axis_index("s")
    wid = sid * 2 + cid
    b = wid // (_NW // _B)
    off = (wid % (_NW // _B)) * _ROWS_PER_W
    pltpu.sync_copy(dl_hbm, dlv)
    pltpu.sync_copy(nw_hbm, nwv)
    iot = lax.iota(jnp.int32, 16)
    dv = dlv[pl.ds(b * _KMAX, 16)]
    wv = nwv[pl.ds(b * _KMAX, 16)]
    out_base = pl.multiple_of(b * _L + off, _CH)

    def step(wk, dk, t0, init):
        for h in range(_CH // 16):
            rows = lax.rem(t0 + h * 16 + dk + iot, _L) + b * _L
            idx[pl.ds(h * 16, 16)] = rows
        pltpu.async_copy(vf_hbm.at[idx], buf, sem).wait()

        def body(i, _):
            def inner(j, _):
                sl = pl.ds(j * 16, 16)
                if init:
                    acc[i, sl] = wk * buf[i, sl]
                else:
                    acc[i, sl] = acc[i, sl] + wk * buf[i, sl]
                return 0
            return lax.fori_loop(0, _HD // 16, inner, 0)
        lax.fori_loop(0, _CH, body, 0)

    def chunk(c, _):
        t0 = off + c * _CH
        for k in range(_KMAX):
            wk = wv[k]
            dk = dv[k]
            if k == 0:
                step(wk, dk, t0, True)
            else:
                pl.when(wk != 0.0)(
                    functools.partial(step, wk, dk, t0, False))
        orow = pl.multiple_of(out_base + c * _CH, _CH)
        pltpu.sync_copy(acc, out_hbm.at[pl.ds(orow, _CH), :])
        return 0

    lax.fori_loop(0, _NCHUNK, chunk, 0)


def _aggregate(vf, delays, nw):
    mesh = plsc.VectorSubcoreMesh(core_axis_name="c", subcore_axis_name="s")
    kern = functools.partial(
        pl.kernel,
        mesh=mesh,
        out_type=jax.ShapeDtypeStruct((_B * _L, _HD), jnp.float32),
        scratch_types=[
            pltpu.VMEM((_B * _KMAX,), jnp.int32),
            pltpu.VMEM((_B * _KMAX,), jnp.float32),
            pltpu.VMEM((_CH,), jnp.int32),
            pltpu.VMEM((_CH, _HD), jnp.float32),
            pltpu.VMEM((_CH, _HD), jnp.float32),
            pltpu.SemaphoreType.DMA,
        ],
    )(_agg_sc_kernel)
    return kern(vf, delays, nw)


def kernel(queries, keys, values, attn_mask, scale_weights, frequency_filter):
    B, Lq, H, E = queries.shape
    D = values.shape[-1]
    qn, kn = _layer_norm(queries, keys)
    qt = qn.transpose(0, 2, 3, 1).reshape(B * H * E, Lq)
    kt = kn.transpose(0, 2, 3, 1).reshape(B * H * E, Lq)

    ff = jax.nn.sigmoid(frequency_filter[0]).reshape(1, 1)
    svec = _scale_vec(scale_weights)
    mean_sum, energy = _correlate(qt, kt, svec, ff)
    mean_corr = mean_sum / (H * E)

    se = jnp.sort(energy, axis=-1)[:, ::-1]
    fd = se[:, :-1] - se[:, 1:]
    sd = fd[:, :-1] - fd[:, 1:]
    elbow = jnp.argmax(sd, axis=-1) + 2
    min_k = max(2, int(0.1 * math.log(Lq)))
    max_k = min(int(0.3 * Lq), int(math.log(Lq) * 2))
    if min_k > max_k:
        max_k = min_k
    ak = jnp.clip(elbow, min_k, max_k).astype(jnp.float32)
    srt = jnp.sort(ak)
    top_k = srt[(srt.shape[0] - 1) // 2].astype(jnp.int32)

    weights, delays = lax.top_k(mean_corr, _KMAX)
    active = jnp.arange(_KMAX) < top_k
    masked = jnp.where(active[None, :], weights, -jnp.inf)
    nw = jax.nn.softmax(masked, axis=-1)

    vf = values.reshape(B * Lq, H * D)
    out = _aggregate(vf, delays.astype(jnp.int32).reshape(-1), nw.reshape(-1))
    return out.reshape(B, Lq, H, D)


# speedup vs baseline: 1.1184x; 1.0840x over previous
import functools
import math

import numpy as np

import jax
import jax.numpy as jnp
from jax import lax
from jax.experimental import pallas as pl
from jax.experimental.pallas import tpu as pltpu
from jax.experimental.pallas import tpu_sc as plsc

_B, _L, _H, _E, _D = 2, 4096, 16, 64, 64
_SCALES = (1, 2, 4)
_EPS = 1e-8
_KMAX = 16


def _build_dft():
    groups = []
    col = 0
    for s in _SCALES:
        Lc = _L // s
        F = Lc // 2 + 1
        Fpad = ((F + 127) // 128) * 128
        groups.append((col, F, Fpad, s))
        col += Fpad
    Ftot = ((col + 127) // 128) * 128
    CF = np.zeros((_L, Ftot), np.float64)
    SF = np.zeros((_L, Ftot), np.float64)
    DI = np.zeros((Ftot, _L), np.float64)
    EI = np.zeros((Ftot, _L), np.float64)
    for (c0, F, Fpad, s) in groups:
        Lc = _L // s
        n = np.arange(Lc, dtype=np.float64)[:, None]
        k = np.arange(F, dtype=np.float64)[None, :]
        ang = 2.0 * np.pi * n * k / Lc
        CF[:, c0:c0 + F] = np.repeat(np.cos(ang) / s, s, axis=0)
        SF[:, c0:c0 + F] = np.repeat(-np.sin(ang) / s, s, axis=0)
        a = np.full((F,), 2.0)
        a[0] = 1.0
        a[-1] = 1.0
        kk = np.arange(F, dtype=np.float64)[:, None]
        nn = np.arange(Lc, dtype=np.float64)[None, :]
        ang2 = 2.0 * np.pi * kk * nn / Lc
        di0 = a[:, None] * np.cos(ang2) / Lc
        ei0 = -a[:, None] * np.sin(ang2) / Lc
        if s == 1:
            DI[c0:c0 + F, :] = di0
            EI[c0:c0 + F, :] = ei0
        else:
            i = np.arange(_L, dtype=np.float64)
            src = np.maximum((i + 0.5) * (Lc / _L) - 0.5, 0.0)
            i0 = np.clip(np.floor(src).astype(np.int64), 0, Lc - 1)
            i1 = np.clip(i0 + 1, 0, Lc - 1)
            w = src - i0
            DI[c0:c0 + F, :] = di0[:, i0] * (1.0 - w) + di0[:, i1] * w
            EI[c0:c0 + F, :] = ei0[:, i0] * (1.0 - w) + ei0[:, i1] * w
    FT = 128
    nt = Ftot // FT
    FB = np.concatenate(
        [CF.reshape(_L, nt, FT), SF.reshape(_L, nt, FT)], axis=2
    ).reshape(_L, 2 * Ftot).astype(np.float32)
    IB = np.concatenate(
        [DI.reshape(nt, FT, _L), EI.reshape(nt, FT, _L)], axis=1
    ).reshape(2 * Ftot, _L).astype(np.float32)
    return FB, IB, groups, Ftot, FT


_FB, _IB, _GROUPS, _FTOT, _FT = _build_dft()
_NT = _FTOT // _FT


def _scale_vec(scale_weights):
    sw = jax.nn.softmax(scale_weights[: len(_SCALES)])
    parts = []
    for gi, (c0, F, Fpad, s) in enumerate(_GROUPS):
        parts.append(jnp.full((Fpad,), sw[gi], jnp.float32))
    v = jnp.concatenate(parts)
    v = jnp.pad(v, (0, _FTOT - v.shape[0]))
    return v[None, :]


def _ln_kernel(q_ref, k_ref, qo_ref, ko_ref):
    for src, dst in ((q_ref, qo_ref), (k_ref, ko_ref)):
        x = src[0]
        m = jnp.mean(x, axis=-1, keepdims=True)
        v = jnp.mean((x - m) ** 2, axis=-1, keepdims=True)
        dst[0] = (x - m) / jnp.sqrt(v + 1e-5)


def _layer_norm(q, k):
    LB = 512
    grid = (_B, _L // LB)
    spec = pl.BlockSpec((1, LB, _H, _E), lambda b, lb: (b, lb, 0, 0))
    out = pl.pallas_call(
        _ln_kernel,
        grid=grid,
        in_specs=[spec, spec],
        out_specs=[spec, spec],
        out_shape=[
            jax.ShapeDtypeStruct(q.shape, jnp.float32),
            jax.ShapeDtypeStruct(k.shape, jnp.float32),
        ],
    )(q, k)
    return out


_CBLK = 256
_G0 = (_B * _H * _E) // _CBLK
_HPB = _CBLK // _E


def _corr_kernel(ff_ref, qt_ref, kt_ref, fb_ref, ib_ref, sv_ref,
                 mean_ref, energy_ref, acc_ref):
    g1 = pl.program_id(1)
    ff = ff_ref[0, 0]
    qf = jnp.dot(qt_ref[...], fb_ref[...], preferred_element_type=jnp.float32)
    kf = jnp.dot(kt_ref[...], fb_ref[...], preferred_element_type=jnp.float32)
    qre, qim = qf[:, :_FT], qf[:, _FT:]
    ure, uim = kf[:, :_FT] * ff, kf[:, _FT:] * ff
    mag = jnp.sqrt(ure * ure + uim * uim)
    inv = (ff * sv_ref[...]) / (mag + _EPS)
    cr = (qre * ure + qim * uim) * inv
    ci = (qim * ure - qre * uim) * inv
    contrib = jnp.dot(
        jnp.concatenate([cr, ci], axis=1), ib_ref[...],
        preferred_element_type=jnp.float32)

    @pl.when(g1 == 0)
    def _():
        acc_ref[...] = contrib

    @pl.when(g1 > 0)
    def _():
        acc_ref[...] = acc_ref[...] + contrib

    @pl.when(g1 == _NT - 1)
    def _():
        corr = jnp.clip(acc_ref[...], -10.0, 10.0)
        msum = jnp.sum(corr, axis=0, keepdims=True)
        esum = jnp.sum(corr * corr, axis=0, keepdims=True)
        g0 = pl.program_id(0)

        @pl.when(g0 % (_G0 // _B) == 0)
        def _():
            mean_ref[0] = msum
            energy_ref[0] = esum

        @pl.when(g0 % (_G0 // _B) != 0)
        def _():
            mean_ref[0] = mean_ref[0] + msum
            energy_ref[0] = energy_ref[0] + esum


def _correlate(qt, kt, svec, ff):
    grid = (_G0, _NT)
    out = pl.pallas_call(
        _corr_kernel,
        grid=grid,
        in_specs=[
            pl.BlockSpec(memory_space=pltpu.SMEM),
            pl.BlockSpec((_CBLK, _L), lambda g0, g1: (g0, 0)),
            pl.BlockSpec((_CBLK, _L), lambda g0, g1: (g0, 0)),
            pl.BlockSpec((_L, 2 * _FT), lambda g0, g1: (0, g1)),
            pl.BlockSpec((2 * _FT, _L), lambda g0, g1: (g1, 0)),
            pl.BlockSpec((1, _FT), lambda g0, g1: (0, g1)),
        ],
        out_specs=[
            pl.BlockSpec((1, 1, _L), lambda g0, g1: (g0 // (_G0 // _B), 0, 0)),
            pl.BlockSpec((1, 1, _L), lambda g0, g1: (g0 // (_G0 // _B), 0, 0)),
        ],
        out_shape=[
            jax.ShapeDtypeStruct((_B, 1, _L), jnp.float32),
            jax.ShapeDtypeStruct((_B, 1, _L), jnp.float32),
        ],
        scratch_shapes=[pltpu.VMEM((_CBLK, _L), jnp.float32)],
    )(ff, qt, kt, jnp.asarray(_FB), jnp.asarray(_IB), svec)
    return out[0][:, 0, :], out[1][:, 0, :]


_HD = _H * _D
_NW = 32
_ROWS_PER_W = (_B * _L) // _NW
_CH = 32
_NCHUNK = _ROWS_PER_W // _CH


def _agg_sc_kernel(vf_hbm, dl_hbm, nw_hbm, out_hbm,
                   dlv, nwv, idx, buf, acc, sem):
    cid = lax.axis_index("c")
    sid = lax.axis_index("s")
    wid = sid * 2 + cid
    b = wid // (_NW // _B)
    off = (wid % (_NW // _B)) * _ROWS_PER_W
    pltpu.sync_copy(dl_hbm, dlv)
    pltpu.sync_copy(nw_hbm, nwv)
    iot = lax.iota(jnp.int32, 16)
    dv = dlv[pl.ds(b * _KMAX, 16)]
    wv = nwv[pl.ds(b * _KMAX, 16)]
    out_base = pl.multiple_of(b * _L + off, _CH)

    def step(wk, dk, t0, init):
        for h in range(_CH // 16):
            rows = lax.rem(t0 + h * 16 + dk + iot, _L) + b * _L
            idx[pl.ds(h * 16, 16)] = rows
        pltpu.async_copy(vf_hbm.at[idx], buf, sem).wait()

        def body(i, _):
            def inner(j, _):
                for u in range(4):
                    sl = pl.ds(j * 64 + u * 16, 16)
                    if init:
                        acc[i, sl] = wk * buf[i, sl]
                    else:
                        acc[i, sl] = acc[i, sl] + wk * buf[i, sl]
                return 0
            return lax.fori_loop(0, _HD // 64, inner, 0)
        lax.fori_loop(0, _CH, body, 0)

    def chunk(c, _):
        t0 = off + c * _CH
        for k in range(_KMAX):
            wk = wv[k]
            dk = dv[k]
            if k == 0:
                step(wk, dk, t0, True)
            else:
                pl.when(wk != 0.0)(
                    functools.partial(step, wk, dk, t0, False))
        orow = pl.multiple_of(out_base + c * _CH, _CH)
        pltpu.sync_copy(acc, out_hbm.at[pl.ds(orow, _CH), :])
        return 0

    lax.fori_loop(0, _NCHUNK, chunk, 0)


def _aggregate(vf, delays, nw):
    mesh = plsc.VectorSubcoreMesh(core_axis_name="c", subcore_axis_name="s")
    kern = functools.partial(
        pl.kernel,
        mesh=mesh,
        out_type=jax.ShapeDtypeStruct((_B * _L, _HD), jnp.float32),
        scratch_types=[
            pltpu.VMEM((_B * _KMAX,), jnp.int32),
            pltpu.VMEM((_B * _KMAX,), jnp.float32),
            pltpu.VMEM((_CH,), jnp.int32),
            pltpu.VMEM((_CH, _HD), jnp.float32),
            pltpu.VMEM((_CH, _HD), jnp.float32),
            pltpu.SemaphoreType.DMA,
        ],
    )(_agg_sc_kernel)
    return kern(vf, delays, nw)


def kernel(queries, keys, values, attn_mask, scale_weights, frequency_filter):
    B, Lq, H, E = queries.shape
    D = values.shape[-1]
    qn, kn = _layer_norm(queries, keys)
    qt = qn.transpose(0, 2, 3, 1).reshape(B * H * E, Lq)
    kt = kn.transpose(0, 2, 3, 1).reshape(B * H * E, Lq)

    ff = jax.nn.sigmoid(frequency_filter[0]).reshape(1, 1)
    svec = _scale_vec(scale_weights)
    mean_sum, energy = _correlate(qt, kt, svec, ff)
    mean_corr = mean_sum / (H * E)

    se = jnp.sort(energy, axis=-1)[:, ::-1]
    fd = se[:, :-1] - se[:, 1:]
    sd = fd[:, :-1] - fd[:, 1:]
    elbow = jnp.argmax(sd, axis=-1) + 2
    min_k = max(2, int(0.1 * math.log(Lq)))
    max_k = min(int(0.3 * Lq), int(math.log(Lq) * 2))
    if min_k > max_k:
        max_k = min_k
    ak = jnp.clip(elbow, min_k, max_k).astype(jnp.float32)
    srt = jnp.sort(ak)
    top_k = srt[(srt.shape[0] - 1) // 2].astype(jnp.int32)

    weights, delays = lax.top_k(mean_corr, _KMAX)
    active = jnp.arange(_KMAX) < top_k
    masked = jnp.where(active[None, :], weights, -jnp.inf)
    nw = jax.nn.softmax(masked, axis=-1)

    vf = values.reshape(B * Lq, H * D)
    out = _aggregate(vf, delays.astype(jnp.int32).reshape(-1), nw.reshape(-1))
    return out.reshape(B, Lq, H, D)
